# baseline (speedup 1.0000x reference)
import functools

import jax
import jax.numpy as jnp
from jax import lax
from jax.experimental import pallas as pl
from jax.experimental.pallas import tpu as pltpu
from jax.experimental.pallas import tpu_sc as plsc

BETA = 0.25
D = 64
K = 1024
HW = 1024
B = 16
BPG = 2
GR = 8
NC, NS = 2, 16
NW = NC * NS
POS_PER_W = B * HW // NW


def _argmin_body(lat_ref, cb_ref, idx_ref, loss_ref):
    c = cb_ref[...]
    c2 = c + c
    b = jnp.sum(c * c, axis=1, keepdims=True)
    part = jnp.float32(0.0)
    for i in range(BPG):
        xb = lat_ref[i]
        a = jnp.sum(xb * xb, axis=0, keepdims=True)
        mm2 = jax.lax.dot_general(
            c2, xb, (((1,), (0,)), ((), ())),
            preferred_element_type=jnp.float32)
        cur = (a + b[0:GR]) - mm2[0:GR]
        curg = jnp.zeros((GR, HW), jnp.int32)
        for g in range(1, K // GR):
            dg = (a + b[g * GR:(g + 1) * GR]) - mm2[g * GR:(g + 1) * GR]
            mask = dg < cur
            cur = jnp.where(mask, dg, cur)
            curg = jnp.where(mask, jnp.int32(g), curg)
        code = curg * GR + jax.lax.broadcasted_iota(jnp.int32, (GR, HW), 0)
        v, cd = cur, code
        sh = GR // 2
        while sh >= 1:
            vlo, vhi = v[:sh], v[sh:]
            clo, chi = cd[:sh], cd[sh:]
            take = (vhi < vlo) | ((vhi == vlo) & (chi < clo))
            v = jnp.where(take, vhi, vlo)
            cd = jnp.where(take, chi, clo)
            sh //= 2
        idx_ref[i] = cd
        part = part + jnp.sum(v)
    prev = jnp.where(pl.program_id(0) == 0, 0.0, loss_ref[0, 0])
    loss_ref[0, 0] = prev + part


def _tc_argmin(lat3, codebook):
    grid = B // BPG
    return pl.pallas_call(
        _argmin_body,
        grid=(grid,),
        in_specs=[
            pl.BlockSpec((BPG, D, HW), lambda i: (i, 0, 0)),
            pl.BlockSpec((K, D), lambda i: (0, 0)),
        ],
        out_specs=[
            pl.BlockSpec((BPG, 1, HW), lambda i: (i, 0, 0)),
            pl.BlockSpec(memory_space=pltpu.SMEM, block_shape=(1, 1),
                         index_map=lambda i: (0, 0)),
        ],
        out_shape=[
            jax.ShapeDtypeStruct((B, 1, HW), jnp.int32),
            jax.ShapeDtypeStruct((1, 1), jnp.float32),
        ],
    )(lat3, codebook)


def _make_sc_gather():
    mesh = plsc.VectorSubcoreMesh(core_axis_name="c", subcore_axis_name="s")

    @functools.partial(
        pl.kernel, mesh=mesh,
        compiler_params=pltpu.CompilerParams(use_tc_tiling_on_sc=False,
                                             needs_layout_passes=False),
        out_type=jax.ShapeDtypeStruct((B, D, HW), jnp.float32),
        scratch_types=[
            pltpu.VMEM((K * D,), jnp.float32),
            pltpu.VMEM((POS_PER_W,), jnp.int32),
            pltpu.VMEM((D, POS_PER_W), jnp.float32),
        ],
    )
    def sc_gather(cbt_hbm, idx_hbm, out_hbm, cbt_v, idx_v, rows_v):
        wid = lax.axis_index("s") * NC + lax.axis_index("c")
        halves = HW // POS_PER_W
        bi = wid // halves
        hw0 = (wid % halves) * POS_PER_W
        pltpu.sync_copy(cbt_hbm, cbt_v)
        pltpu.sync_copy(idx_hbm.at[bi, pl.ds(hw0, POS_PER_W)], idx_v)

        def jbody(j, carry):
            idx16 = idx_v[pl.ds(j * 16, 16)]
            for d in range(D):
                vals = plsc.load_gather(cbt_v, [idx16 + jnp.int32(d * K)])
                rows_v[d, pl.ds(j * 16, 16)] = vals
            return carry

        lax.fori_loop(0, POS_PER_W // 16, jbody, jnp.int32(0))
        pltpu.sync_copy(rows_v, out_hbm.at[bi, :, pl.ds(hw0, POS_PER_W)])

    return sc_gather


def kernel(latents, codebook):
    lat3 = latents.reshape(B, D, HW)
    idx3, loss = _tc_argmin(lat3, codebook)
    cbt_flat = jnp.transpose(codebook).reshape(K * D)
    q3 = _make_sc_gather()(cbt_flat, idx3.reshape(B, HW))
    quantized = q3.reshape(latents.shape)
    vq_loss = (1.0 + BETA) * loss[0, 0] / (B * HW * D)
    return quantized, vq_loss


# speedup vs baseline: 1.7997x; 1.7997x over previous
import jax
import jax.numpy as jnp
from jax.experimental import pallas as pl
from jax.experimental.pallas import tpu as pltpu

BETA = 0.25
D = 64
K = 1024
HW = 1024
B = 16
BPG = 2
GR = 8


def _vq_body(lat_ref, cb_ref, out_ref, loss_ref):
    c = cb_ref[...]
    c2 = c + c
    b = jnp.sum(c * c, axis=1, keepdims=True)
    iota = jax.lax.broadcasted_iota(jnp.int32, (K, HW), 0)
    part = jnp.float32(0.0)
    for i in range(BPG):
        xb = lat_ref[i]
        a = jnp.sum(xb * xb, axis=0, keepdims=True)
        mm2 = jax.lax.dot_general(
            c2, xb, (((1,), (0,)), ((), ())),
            preferred_element_type=jnp.float32)
        cur = (a + b[0:GR]) - mm2[0:GR]
        curg = jnp.zeros((GR, HW), jnp.int32)
        for g in range(1, K // GR):
            dg = (a + b[g * GR:(g + 1) * GR]) - mm2[g * GR:(g + 1) * GR]
            mask = dg < cur
            cur = jnp.where(mask, dg, cur)
            curg = jnp.where(mask, jnp.int32(g), curg)
        code = curg * GR + jax.lax.broadcasted_iota(jnp.int32, (GR, HW), 0)
        v, cd = cur, code
        sh = GR // 2
        while sh >= 1:
            vlo, vhi = v[:sh], v[sh:]
            clo, chi = cd[:sh], cd[sh:]
            take = (vhi < vlo) | ((vhi == vlo) & (chi < clo))
            v = jnp.where(take, vhi, vlo)
            cd = jnp.where(take, chi, clo)
            sh //= 2
        onehot = (iota == cd).astype(jnp.float32)
        q = jax.lax.dot_general(
            c, onehot, (((0,), (0,)), ((), ())),
            preferred_element_type=jnp.float32)
        out_ref[i] = q
        part = part + jnp.sum(v)
    prev = jnp.where(pl.program_id(0) == 0, 0.0, loss_ref[0, 0])
    loss_ref[0, 0] = prev + part


def kernel(latents, codebook):
    lat3 = latents.reshape(B, D, HW)
    grid = B // BPG
    q3, loss = pl.pallas_call(
        _vq_body,
        grid=(grid,),
        in_specs=[
            pl.BlockSpec((BPG, D, HW), lambda i: (i, 0, 0)),
            pl.BlockSpec((K, D), lambda i: (0, 0)),
        ],
        out_specs=[
            pl.BlockSpec((BPG, D, HW), lambda i: (i, 0, 0)),
            pl.BlockSpec(memory_space=pltpu.SMEM, block_shape=(1, 1),
                         index_map=lambda i: (0, 0)),
        ],
        out_shape=[
            jax.ShapeDtypeStruct((B, D, HW), jnp.float32),
            jax.ShapeDtypeStruct((1, 1), jnp.float32),
        ],
    )(lat3, codebook)
    quantized = q3.reshape(latents.shape)
    vq_loss = (1.0 + BETA) * loss[0, 0] / (B * HW * D)
    return quantized, vq_loss


# speedup vs baseline: 1.8132x; 1.0075x over previous
import jax
import jax.numpy as jnp
from jax.experimental import pallas as pl
from jax.experimental.pallas import tpu as pltpu

BETA = 0.25
D = 64
K = 1024
HW = 1024
B = 16
BPG = 4
GR = 8


def _vq_body(lat_ref, cb_ref, out_ref, loss_ref):
    c = cb_ref[...]
    c2 = c + c
    b = jnp.sum(c * c, axis=1, keepdims=True)
    iota = jax.lax.broadcasted_iota(jnp.int32, (K, HW), 0)
    part = jnp.float32(0.0)
    for i in range(BPG):
        xb = lat_ref[i]
        a = jnp.sum(xb * xb, axis=0, keepdims=True)
        mm2 = jax.lax.dot_general(
            c2, xb, (((1,), (0,)), ((), ())),
            preferred_element_type=jnp.float32)
        cur = (a + b[0:GR]) - mm2[0:GR]
        curg = jnp.zeros((GR, HW), jnp.int32)
        for g in range(1, K // GR):
            dg = (a + b[g * GR:(g + 1) * GR]) - mm2[g * GR:(g + 1) * GR]
            mask = dg < cur
            cur = jnp.where(mask, dg, cur)
            curg = jnp.where(mask, jnp.int32(g), curg)
        code = curg * GR + jax.lax.broadcasted_iota(jnp.int32, (GR, HW), 0)
        v, cd = cur, code
        sh = GR // 2
        while sh >= 1:
            vlo, vhi = v[:sh], v[sh:]
            clo, chi = cd[:sh], cd[sh:]
            take = (vhi < vlo) | ((vhi == vlo) & (chi < clo))
            v = jnp.where(take, vhi, vlo)
            cd = jnp.where(take, chi, clo)
            sh //= 2
        onehot = (iota == cd).astype(jnp.float32)
        q = jax.lax.dot_general(
            c, onehot, (((0,), (0,)), ((), ())),
            preferred_element_type=jnp.float32)
        out_ref[i] = q
        part = part + jnp.sum(v)
    prev = jnp.where(pl.program_id(0) == 0, 0.0, loss_ref[0, 0])
    loss_ref[0, 0] = prev + part


def kernel(latents, codebook):
    lat3 = latents.reshape(B, D, HW)
    grid = B // BPG
    q3, loss = pl.pallas_call(
        _vq_body,
        grid=(grid,),
        in_specs=[
            pl.BlockSpec((BPG, D, HW), lambda i: (i, 0, 0)),
            pl.BlockSpec((K, D), lambda i: (0, 0)),
        ],
        out_specs=[
            pl.BlockSpec((BPG, D, HW), lambda i: (i, 0, 0)),
            pl.BlockSpec(memory_space=pltpu.SMEM, block_shape=(1, 1),
                         index_map=lambda i: (0, 0)),
        ],
        out_shape=[
            jax.ShapeDtypeStruct((B, D, HW), jnp.float32),
            jax.ShapeDtypeStruct((1, 1), jnp.float32),
        ],
    )(lat3, codebook)
    quantized = q3.reshape(latents.shape)
    vq_loss = (1.0 + BETA) * loss[0, 0] / (B * HW * D)
    return quantized, vq_loss


# speedup vs baseline: 1.8135x; 1.0002x over previous
import jax
import jax.numpy as jnp
from jax.experimental import pallas as pl
from jax.experimental.pallas import tpu as pltpu

BETA = 0.25
D = 64
K = 1024
HW = 1024
B = 16
BPG = 4
GR = 8


def _vq_body(lat_ref, cb_ref, out_ref, loss_ref):
    c = cb_ref[...]
    c2 = c + c
    b = jnp.sum(c * c, axis=1, keepdims=True)
    iota = jax.lax.broadcasted_iota(jnp.int32, (K, HW), 0)
    part = jnp.float32(0.0)
    for i in range(BPG):
        xb = lat_ref[i]
        a = jnp.sum(xb * xb, axis=0, keepdims=True)
        mm2 = jax.lax.dot_general(
            c2, xb, (((1,), (0,)), ((), ())),
            preferred_element_type=jnp.float32)
        cur = (a + b[0:GR]) - mm2[0:GR]
        curg = jnp.zeros((GR, HW), jnp.int32)
        for g in range(1, K // GR):
            dg = (a + b[g * GR:(g + 1) * GR]) - mm2[g * GR:(g + 1) * GR]
            mask = dg < cur
            cur = jnp.where(mask, dg, cur)
            curg = jnp.where(mask, jnp.int32(g), curg)
        code = curg * GR + jax.lax.broadcasted_iota(jnp.int32, (GR, HW), 0)
        v, cd = cur, code
        sh = GR // 2
        while sh >= 1:
            vlo, vhi = v[:sh], v[sh:]
            clo, chi = cd[:sh], cd[sh:]
            take = (vhi < vlo) | ((vhi == vlo) & (chi < clo))
            v = jnp.where(take, vhi, vlo)
            cd = jnp.where(take, chi, clo)
            sh //= 2
        onehot = (iota == cd).astype(jnp.bfloat16)
        q = jax.lax.dot_general(
            c, onehot, (((0,), (0,)), ((), ())),
            preferred_element_type=jnp.float32)
        out_ref[i] = q
        part = part + jnp.sum(v)
    prev = jnp.where(pl.program_id(0) == 0, 0.0, loss_ref[0, 0])
    loss_ref[0, 0] = prev + part


def kernel(latents, codebook):
    lat3 = latents.reshape(B, D, HW)
    grid = B // BPG
    q3, loss = pl.pallas_call(
        _vq_body,
        grid=(grid,),
        in_specs=[
            pl.BlockSpec((BPG, D, HW), lambda i: (i, 0, 0)),
            pl.BlockSpec((K, D), lambda i: (0, 0)),
        ],
        out_specs=[
            pl.BlockSpec((BPG, D, HW), lambda i: (i, 0, 0)),
            pl.BlockSpec(memory_space=pltpu.SMEM, block_shape=(1, 1),
                         index_map=lambda i: (0, 0)),
        ],
        out_shape=[
            jax.ShapeDtypeStruct((B, D, HW), jnp.float32),
            jax.ShapeDtypeStruct((1, 1), jnp.float32),
        ],
    )(lat3, codebook)
    quantized = q3.reshape(latents.shape)
    vq_loss = (1.0 + BETA) * loss[0, 0] / (B * HW * D)
    return quantized, vq_loss
